# Initial kernel scaffold; baseline (speedup 1.0000x reference)
import jax
import jax.numpy as jnp
from jax.experimental import pallas as pl


def kernel(observations, edge_index, params):
    raise NotImplementedError("write your pallas kernel here")


# speedup vs baseline: 4.8669x; 4.8669x over previous
import numpy as np
import jax
import jax.numpy as jnp
from jax.experimental import pallas as pl
from jax.experimental.pallas import tpu as pltpu

N_CELLS = 64
F_PER_CELL = 16
D = 64
BAND = 8
BB = 16

GROUPS = tuple((o, s) for s in (1, -1) for o in range(1, BAND + 1))


def _dot(a, b):
    return jnp.dot(a, b, preferred_element_type=jnp.float32)


def _shift(x, o, sgn):
    z = jnp.zeros((o, x.shape[1]), x.dtype)
    if sgn > 0:
        return jnp.concatenate([x[o:], z], axis=0)
    return jnp.concatenate([z, x[: x.shape[0] - o]], axis=0)


def _ln(x, g, b):
    m = jnp.mean(x, axis=1, keepdims=True)
    xc = x - m
    v = jnp.mean(xc * xc, axis=1, keepdims=True)
    return xc * jax.lax.rsqrt(v + 1e-5) * g + b


def _fwd_kernel(
    x_ref,
    emb_W, emb_b,
    e0_W1a, e0_W1b, e0_b1, e0_W2, e0_b2,
    n0_W1a, n0_W1b, n0_b1, n0_W2, n0_b2,
    nn_g0, nn_b0, en_g0, en_b0,
    e1_W1a, e1_W1b, e1_W1c, e1_b1, e1_W2, e1_b2,
    n1_W1a, n1_W1b, n1_b1, n1_W2, n1_b2,
    nn_g1, nn_b1,
    h_Wa, h_Wb, h_b, o_W, o_b,
    out_ref,
    e_scr,
):
    R = BB * N_CELLS
    jcell = jax.lax.broadcasted_iota(jnp.int32, (R, 1), 0) % N_CELLS

    def band_mask(o, sgn):
        return (jcell < N_CELLS - o) if sgn > 0 else (jcell >= o)

    node = jnp.maximum(_dot(x_ref[...], emb_W[...]) + emb_b[...], 0.0)

    P = _dot(node, e0_W1a[...])
    Q = _dot(node, e0_W1b[...]) + e0_b1[...]
    W2, b2 = e0_W2[...], e0_b2[...]
    eg, eb = en_g0[...], en_b0[...]
    agg = jnp.zeros((R, D), jnp.float32)
    for gi, (o, sgn) in enumerate(GROUPS):
        h = jnp.maximum(_shift(P, o, sgn) + Q, 0.0)
        e = _dot(h, W2) + b2
        agg = agg + jnp.where(band_mask(o, sgn), e, 0.0)
        e_scr[gi] = _ln(e, eg, eb)
    h = jnp.maximum(_dot(node, n0_W1a[...]) + _dot(agg, n0_W1b[...]) + n0_b1[...], 0.0)
    node = _ln(_dot(h, n0_W2[...]) + n0_b2[...], nn_g0[...], nn_b0[...])

    P = _dot(node, e1_W1a[...])
    Q = _dot(node, e1_W1b[...]) + e1_b1[...]
    W1c, W2, b2 = e1_W1c[...], e1_W2[...], e1_b2[...]
    agg = jnp.zeros((R, D), jnp.float32)
    for gi, (o, sgn) in enumerate(GROUPS):
        h = jnp.maximum(_shift(P, o, sgn) + Q + _dot(e_scr[gi], W1c), 0.0)
        e = _dot(h, W2) + b2
        agg = agg + jnp.where(band_mask(o, sgn), e, 0.0)
    h = jnp.maximum(_dot(node, n1_W1a[...]) + _dot(agg, n1_W1b[...]) + n1_b1[...], 0.0)
    node = _ln(_dot(h, n1_W2[...]) + n1_b2[...], nn_g1[...], nn_b1[...])

    Ph = _dot(node, h_Wa[...])
    Qh = _dot(node, h_Wb[...]) + h_b[...]
    oW, ob = o_W[...], o_b[...]
    cols = []
    for o in range(1, BAND + 1):
        hid = jnp.maximum(_shift(Ph, o, 1) + Qh, 0.0)
        cols.append(jnp.tanh(_dot(hid, oW) + ob))
    out_ref[...] = jnp.concatenate(cols, axis=1)


def _pair_select():
    sel = []
    for i in range(N_CELLS):
        for j in range(max(0, i - BAND), i):
            sel.append(j * BAND + (i - j - 1))
    return np.asarray(sel, dtype=np.int32)


_SEL = _pair_select()


def kernel(observations, edge_index, params):
    B = observations.shape[0]
    x = observations.reshape(B * N_CELLS, F_PER_CELL)
    p = params
    g0, g1 = p['gnn'][0], p['gnn'][1]
    r2 = lambda v: v.reshape(1, -1)
    e0W1, e1W1 = g0['eW'][0], g1['eW'][0]
    n0W1, n1W1 = g0['nW'][0], g1['nW'][0]
    hW = p['hid_W']
    weights = [
        p['emb_W'], r2(p['emb_b']),
        e0W1[:D], e0W1[D:], r2(g0['eb'][0]), g0['eW'][1], r2(g0['eb'][1]),
        n0W1[:D], n0W1[D:], r2(g0['nb'][0]), g0['nW'][1], r2(g0['nb'][1]),
        r2(p['nn_g'][0]), r2(p['nn_b'][0]), r2(p['en_g'][0]), r2(p['en_b'][0]),
        e1W1[:D], e1W1[D:2 * D], e1W1[2 * D:], r2(g1['eb'][0]), g1['eW'][1], r2(g1['eb'][1]),
        n1W1[:D], n1W1[D:], r2(g1['nb'][0]), g1['nW'][1], r2(g1['nb'][1]),
        r2(p['nn_g'][1]), r2(p['nn_b'][1]),
        hW[:D], hW[D:], r2(p['hid_b']), p['out_W'], r2(p['out_b']),
    ]
    R = BB * N_CELLS
    out = pl.pallas_call(
        _fwd_kernel,
        grid=(B // BB,),
        in_specs=[pl.BlockSpec((R, F_PER_CELL), lambda i: (i, 0))]
        + [pl.BlockSpec(w.shape, lambda i, nd=w.ndim: (0,) * nd) for w in weights],
        out_specs=pl.BlockSpec((R, BAND), lambda i: (i, 0)),
        out_shape=jax.ShapeDtypeStruct((B * N_CELLS, BAND), jnp.float32),
        scratch_shapes=[pltpu.VMEM((len(GROUPS), R, D), jnp.float32)],
    )(x, *weights)
    return out.reshape(B, N_CELLS * BAND)[:, _SEL]


# speedup vs baseline: 5.8562x; 1.2033x over previous
import numpy as np
import jax
import jax.numpy as jnp
from jax.experimental import pallas as pl
from jax.experimental.pallas import tpu as pltpu

N_CELLS = 64
F_PER_CELL = 16
D = 64
BAND = 8
BH = 8
RH = BH * N_CELLS
NEG = -1e30

GROUPS = tuple((o, s) for s in (1, -1) for o in range(1, BAND + 1))


def _dot(a, b):
    return jnp.dot(a, b, preferred_element_type=jnp.float32)


def _ln(x, g, b, onebd):
    m = _dot(x, onebd)
    ms = _dot(x * x, onebd)
    r = jax.lax.rsqrt(ms - m * m + 1e-5)
    return (x - m) * r * g + b


def _fwd_kernel(
    x_ref,
    emb_W, emb_b,
    e0_W1a, e0_W1b, e0_b1, e0_W2, e0_b2,
    n0_W1a, n0_W1b, n0_b1, n0_W2, n0_b2,
    nn_g0, nn_b0,
    e1_W1a, e1_W1b, e1_b1, e1_W2, e1_b2, Wfus, bfus, cs,
    n1_W1a, n1_W1b, n1_b1, n1_W2, n1_b2,
    nn_g1, nn_b1,
    h_Wa, h_Wb, h_b, o_W, o_b, onebd,
    out_ref,
    h_scr, p_scr,
):
    DL = 2 * D
    p_scr[:, :BAND, :] = jnp.full((BH, BAND, DL), NEG, jnp.float32)
    p_scr[:, BAND + N_CELLS:, :] = jnp.full((BH, BAND, DL), NEG, jnp.float32)

    def put(P):
        p_scr[:, BAND:BAND + N_CELLS, :] = P.reshape(BH, N_CELLS, DL)

    def shifted(o, sgn):
        d = BAND + sgn * o
        return p_scr[:, d:d + N_CELLS, :].reshape(RH, DL)

    onebdv = onebd[...]
    jcell = jax.lax.broadcasted_iota(jnp.int32, (RH, 1), 0) % N_CELLS
    deg = (jnp.minimum(jcell, BAND) + jnp.minimum(N_CELLS - 1 - jcell, BAND)
           ).astype(jnp.float32)

    node = jnp.maximum(_dot(x_ref[...], emb_W[...]) + emb_b[...], 0.0)

    put(_dot(node, e0_W1a[...]))
    Q = _dot(node, e0_W1b[...]) + e0_b1[...]
    Wf, bf, csv = Wfus[...], bfus[...], cs[...]
    Hs = jnp.zeros((RH, DL), jnp.float32)
    for gi, (o, sgn) in enumerate(GROUPS):
        h = jnp.maximum(shifted(o, sgn) + Q, 0.0)
        Hs = Hs + h
        ECM = _dot(h, Wf) + bf
        e, Cc, mw = ECM[:, :DL], ECM[:, DL:2 * DL], ECM[:, 2 * DL:]
        msw = _dot(e * e, onebdv)
        rw = jax.lax.rsqrt(msw - mw * mw + 1e-5)
        h_scr[gi] = Cc * rw - (mw * rw) * csv
    agg = _dot(Hs, e0_W2[...]) + deg * e0_b2[...]
    h = jnp.maximum(_dot(node, n0_W1a[...]) + _dot(agg, n0_W1b[...]) + n0_b1[...], 0.0)
    node = _ln(_dot(h, n0_W2[...]) + n0_b2[...], nn_g0[...], nn_b0[...], onebdv)

    put(_dot(node, e1_W1a[...]))
    Q = _dot(node, e1_W1b[...]) + e1_b1[...]
    Hs = jnp.zeros((RH, DL), jnp.float32)
    for gi, (o, sgn) in enumerate(GROUPS):
        Hs = Hs + jnp.maximum(shifted(o, sgn) + Q + h_scr[gi], 0.0)
    agg = _dot(Hs, e1_W2[...]) + deg * e1_b2[...]
    h = jnp.maximum(_dot(node, n1_W1a[...]) + _dot(agg, n1_W1b[...]) + n1_b1[...], 0.0)
    node = _ln(_dot(h, n1_W2[...]) + n1_b2[...], nn_g1[...], nn_b1[...], onebdv)

    put(_dot(node, h_Wa[...]))
    Qh = _dot(node, h_Wb[...]) + h_b[...]
    oW = o_W[...]
    cols = []
    for o in range(1, BAND + 1):
        hid = jnp.maximum(shifted(o, 1) + Qh, 0.0)
        cols.append(_dot(hid, oW))
    out_ref[...] = jnp.tanh(jnp.concatenate(cols, axis=1) + o_b[...])


def _pair_select():
    sel = []
    for i in range(N_CELLS):
        for j in range(max(0, i - BAND), i):
            sel.append(j * BAND + (i - j - 1))
    return np.asarray(sel, dtype=np.int32)


_SEL = _pair_select()


def _bd(W):
    z = jnp.zeros_like(W)
    return jnp.concatenate(
        [jnp.concatenate([W, z], axis=1), jnp.concatenate([z, W], axis=1)],
        axis=0)


def _t2(v):
    return jnp.concatenate([v.reshape(1, -1), v.reshape(1, -1)], axis=1)


def kernel(observations, edge_index, params):
    B = observations.shape[0]
    CH = B // (2 * BH)
    x = observations.reshape(CH, 2, BH, N_CELLS, F_PER_CELL)
    x = x.transpose(0, 2, 3, 1, 4).reshape(CH * RH, 2 * F_PER_CELL)
    p = params
    g0, g1 = p['gnn'][0], p['gnn'][1]
    e0W1, e1W1 = g0['eW'][0], g1['eW'][0]
    n0W1, n1W1 = g0['nW'][0], g1['nW'][0]
    hW = p['hid_W']
    W1c = e1W1[2 * D:]
    W1cp = p['en_g'][0][:, None] * W1c
    cs = W1cp.sum(axis=0)
    W2_0, b2_0 = g0['eW'][1], g0['eb'][1]
    onesD = jnp.full((D, D), 1.0 / D, jnp.float32)
    Wfus = jnp.concatenate(
        [_bd(W2_0), _bd(W2_0 @ W1cp), _bd(W2_0 @ onesD)], axis=1)
    bfus = jnp.concatenate([_t2(b2_0), _t2(b2_0 @ W1cp), _t2(b2_0 @ onesD)],
                           axis=1)
    e1_b1_eff = g1['eb'][0] + p['en_b'][0] @ W1c
    weights = [
        _bd(p['emb_W']), _t2(p['emb_b']),
        _bd(e0W1[:D]), _bd(e0W1[D:]), _t2(g0['eb'][0]), _bd(g0['eW'][1]),
        _t2(g0['eb'][1]),
        _bd(n0W1[:D]), _bd(n0W1[D:]), _t2(g0['nb'][0]), _bd(g0['nW'][1]),
        _t2(g0['nb'][1]),
        _t2(p['nn_g'][0]), _t2(p['nn_b'][0]),
        _bd(e1W1[:D]), _bd(e1W1[D:2 * D]), _t2(e1_b1_eff), _bd(g1['eW'][1]),
        _t2(g1['eb'][1]),
        Wfus, bfus, _t2(cs),
        _bd(n1W1[:D]), _bd(n1W1[D:]), _t2(g1['nb'][0]), _bd(g1['nW'][1]),
        _t2(g1['nb'][1]),
        _t2(p['nn_g'][1]), _t2(p['nn_b'][1]),
        _bd(hW[:D]), _bd(hW[D:]), _t2(p['hid_b']), _bd(p['out_W']),
        p['out_b'].reshape(1, 1), _bd(onesD),
    ]
    out = pl.pallas_call(
        _fwd_kernel,
        grid=(CH,),
        in_specs=[pl.BlockSpec((RH, 2 * F_PER_CELL), lambda i: (i, 0))]
        + [pl.BlockSpec(w.shape, lambda i, nd=w.ndim: (0,) * nd) for w in weights],
        out_specs=pl.BlockSpec((RH, 2 * BAND), lambda i: (i, 0)),
        out_shape=jax.ShapeDtypeStruct((CH * RH, 2 * BAND), jnp.float32),
        scratch_shapes=[
            pltpu.VMEM((len(GROUPS), RH, 2 * D), jnp.float32),
            pltpu.VMEM((BH, N_CELLS + 2 * BAND, 2 * D), jnp.float32),
        ],
    )(x, *weights)
    out = out.reshape(CH, BH, N_CELLS, BAND, 2)
    out = out.transpose(0, 4, 1, 2, 3).reshape(B, N_CELLS * BAND)
    return out[:, _SEL]


# speedup vs baseline: 6.2067x; 1.0598x over previous
import numpy as np
import jax
import jax.numpy as jnp
from jax.experimental import pallas as pl
from jax.experimental.pallas import tpu as pltpu

N_CELLS = 64
F_PER_CELL = 16
D = 64
BAND = 8
BH = 8
RH = BH * N_CELLS
NEG = -1e30

GROUPS = tuple((o, s) for s in (1, -1) for o in range(1, BAND + 1))


def _dot(a, b):
    return jnp.dot(a, b, preferred_element_type=jnp.float32)


def _ln(x, g, b, onebd):
    m = _dot(x, onebd)
    ms = _dot(x * x, onebd)
    r = jax.lax.rsqrt(ms - m * m + 1e-5)
    return (x - m) * r * g + b


def _fwd_kernel(
    x_ref,
    emb_W, emb_b,
    e0_W1ab, e0_b1, e0_W2, degb0,
    n0_W1a, n0_W1b, n0_b1, n0_W2, n0_b2,
    nn_g0, nn_b0,
    e1_W1ab, e1_b1, e1_W2, degb1, Wfus, bfus,
    n1_W1a, n1_W1b, n1_b1, n1_W2, n1_b2,
    nn_g1, nn_b1,
    h_Wab, h_b, o_W, o_b, onebd, one2bd,
    out_ref,
    h_scr, p_scr,
):
    DL = 2 * D

    @pl.when(pl.program_id(0) == 0)
    def _init_pads():
        p_scr[:, :BAND, :] = jnp.full((BH, BAND, DL), NEG, jnp.float32)
        p_scr[:, BAND + N_CELLS:, :] = jnp.full((BH, BAND, DL), NEG,
                                                jnp.float32)

    def put(P):
        p_scr[:, BAND:BAND + N_CELLS, :] = P.reshape(BH, N_CELLS, DL)

    def shifted(o, sgn):
        d = BAND + sgn * o
        return p_scr[:, d:d + N_CELLS, :].reshape(RH, DL)

    onebdv = onebd[...]

    node = jnp.maximum(_dot(x_ref[...], emb_W[...]) + emb_b[...], 0.0)

    PQ = _dot(node, e0_W1ab[...])
    put(PQ[:, :DL])
    Q = PQ[:, DL:] + e0_b1[...]
    Wf, bf, o2 = Wfus[...], bfus[...], one2bd[...]
    Hs = jnp.zeros((RH, DL), jnp.float32)
    for gi, (o, sgn) in enumerate(GROUPS):
        h = jnp.maximum(shifted(o, sgn) + Q, 0.0)
        Hs = Hs + h
        EC = _dot(h, Wf) + bf
        e = EC[:, :DL]
        MM = _dot(jnp.concatenate([e, e * e], axis=1), o2)
        mw, msw = MM[:, :DL], MM[:, DL:]
        rw = jax.lax.rsqrt(msw - mw * mw + 1e-5)
        h_scr[gi] = EC[:, DL:] * rw
    agg = _dot(Hs, e0_W2[...]) + degb0[...]
    h = jnp.maximum(_dot(node, n0_W1a[...]) + _dot(agg, n0_W1b[...]) + n0_b1[...], 0.0)
    node = _ln(_dot(h, n0_W2[...]) + n0_b2[...], nn_g0[...], nn_b0[...], onebdv)

    PQ = _dot(node, e1_W1ab[...])
    put(PQ[:, :DL])
    Q = PQ[:, DL:] + e1_b1[...]
    Hs = jnp.zeros((RH, DL), jnp.float32)
    for gi, (o, sgn) in enumerate(GROUPS):
        Hs = Hs + jnp.maximum(shifted(o, sgn) + Q + h_scr[gi], 0.0)
    agg = _dot(Hs, e1_W2[...]) + degb1[...]
    h = jnp.maximum(_dot(node, n1_W1a[...]) + _dot(agg, n1_W1b[...]) + n1_b1[...], 0.0)
    node = _ln(_dot(h, n1_W2[...]) + n1_b2[...], nn_g1[...], nn_b1[...], onebdv)

    PQ = _dot(node, h_Wab[...])
    put(PQ[:, :DL])
    Qh = PQ[:, DL:] + h_b[...]
    oW = o_W[...]
    cols = []
    for o in range(1, BAND + 1):
        hid = jnp.maximum(shifted(o, 1) + Qh, 0.0)
        cols.append(_dot(hid, oW))
    out_ref[...] = jnp.tanh(jnp.concatenate(cols, axis=1) + o_b[...])


def _pair_select():
    sel = []
    for i in range(N_CELLS):
        for j in range(max(0, i - BAND), i):
            sel.append(j * BAND + (i - j - 1))
    return np.asarray(sel, dtype=np.int32)


_SEL = _pair_select()


def _bd(W):
    z = jnp.zeros_like(W)
    return jnp.concatenate(
        [jnp.concatenate([W, z], axis=1), jnp.concatenate([z, W], axis=1)],
        axis=0)


def _t2(v):
    return jnp.concatenate([v.reshape(1, -1), v.reshape(1, -1)], axis=1)


def kernel(observations, edge_index, params):
    B = observations.shape[0]
    CH = B // (2 * BH)
    x = observations.reshape(CH, 2, BH, N_CELLS, F_PER_CELL)
    x = x.transpose(0, 2, 3, 1, 4).reshape(CH * RH, 2 * F_PER_CELL)
    p = params
    g0, g1 = p['gnn'][0], p['gnn'][1]
    e0W1, e1W1 = g0['eW'][0], g1['eW'][0]
    n0W1, n1W1 = g0['nW'][0], g1['nW'][0]
    hW = p['hid_W']
    W1c = e1W1[2 * D:]
    W1cp = p['en_g'][0][:, None] * W1c
    W1cp_eff = W1cp - jnp.full((D, D), 1.0 / D, jnp.float32) * W1cp.sum(axis=0)[None, :]
    W2_0, b2_0 = g0['eW'][1], g0['eb'][1]
    onesD = jnp.full((D, D), 1.0 / D, jnp.float32)
    Wfus = jnp.concatenate([_bd(W2_0), _bd(W2_0 @ W1cp_eff)], axis=1)
    bfus = jnp.concatenate([_t2(b2_0), _t2(b2_0 @ W1cp_eff)], axis=1)
    e1_b1_eff = g1['eb'][0] + p['en_b'][0] @ W1c
    jc = np.arange(RH, dtype=np.int32) % N_CELLS
    deg = (np.minimum(jc, BAND) + np.minimum(N_CELLS - 1 - jc, BAND)
           ).astype(np.float32)[:, None]
    degc = jnp.asarray(deg)
    degb0 = degc * _t2(g0['eb'][1])
    degb1 = degc * _t2(g1['eb'][1])
    onebd = _bd(onesD)
    weights = [
        _bd(p['emb_W']), _t2(p['emb_b']),
        jnp.concatenate([_bd(e0W1[:D]), _bd(e0W1[D:])], axis=1),
        _t2(g0['eb'][0]), _bd(g0['eW'][1]), degb0,
        _bd(n0W1[:D]), _bd(n0W1[D:]), _t2(g0['nb'][0]), _bd(g0['nW'][1]),
        _t2(g0['nb'][1]),
        _t2(p['nn_g'][0]), _t2(p['nn_b'][0]),
        jnp.concatenate([_bd(e1W1[:D]), _bd(e1W1[D:2 * D])], axis=1),
        _t2(e1_b1_eff), _bd(g1['eW'][1]), degb1, Wfus, bfus,
        _bd(n1W1[:D]), _bd(n1W1[D:]), _t2(g1['nb'][0]), _bd(g1['nW'][1]),
        _t2(g1['nb'][1]),
        _t2(p['nn_g'][1]), _t2(p['nn_b'][1]),
        jnp.concatenate([_bd(hW[:D]), _bd(hW[D:])], axis=1),
        _t2(p['hid_b']), _bd(p['out_W']), p['out_b'].reshape(1, 1),
        onebd, _bd(onebd),
    ]
    out = pl.pallas_call(
        _fwd_kernel,
        grid=(CH,),
        in_specs=[pl.BlockSpec((RH, 2 * F_PER_CELL), lambda i: (i, 0))]
        + [pl.BlockSpec(w.shape, lambda i, nd=w.ndim: (0,) * nd) for w in weights],
        out_specs=pl.BlockSpec((RH, 2 * BAND), lambda i: (i, 0)),
        out_shape=jax.ShapeDtypeStruct((CH * RH, 2 * BAND), jnp.float32),
        scratch_shapes=[
            pltpu.VMEM((len(GROUPS), RH, 2 * D), jnp.float32),
            pltpu.VMEM((BH, N_CELLS + 2 * BAND, 2 * D), jnp.float32),
        ],
    )(x, *weights)
    out = out.reshape(CH, BH, N_CELLS, BAND, 2)
    out = out.transpose(0, 4, 1, 2, 3).reshape(B, N_CELLS * BAND)
    return out[:, _SEL]


# speedup vs baseline: 7.2065x; 1.1611x over previous
import numpy as np
import jax
import jax.numpy as jnp
from jax.experimental import pallas as pl
from jax.experimental.pallas import tpu as pltpu

N_CELLS = 64
F_PER_CELL = 16
D = 64
BAND = 8
BH = 16
RH = BH * N_CELLS
NEG = -1e30

GROUPS = tuple((o, s) for s in (1, -1) for o in range(1, BAND + 1))


def _dot(a, b):
    return jnp.dot(a, b, preferred_element_type=jnp.float32)


def _ln(x, g, b, onebd):
    m = _dot(x, onebd)
    ms = _dot(x * x, onebd)
    r = jax.lax.rsqrt(ms - m * m + 1e-5)
    return (x - m) * r * g + b


def _fwd_kernel(
    x_ref,
    emb_W, emb_b,
    e0_W1ab, e0_b1, e0_W2, degb0,
    n0_W1a, n0_W1b, n0_b1, n0_W2, n0_b2,
    nn_g0, nn_b0,
    e1_W1ab, e1_b1, e1_W2, degb1, Wfus, bfus,
    n1_W1a, n1_W1b, n1_b1, n1_W2, n1_b2,
    nn_g1, nn_b1,
    h_Wab, h_b, o_W, o_b, onebd, one2bd,
    out_ref,
    h_scr, p_scr,
):
    DL = 2 * D

    @pl.when(pl.program_id(0) == 0)
    def _init_pads():
        p_scr[:, :BAND, :] = jnp.full((BH, BAND, DL), NEG, jnp.float32)
        p_scr[:, BAND + N_CELLS:, :] = jnp.full((BH, BAND, DL), NEG,
                                                jnp.float32)

    def put(P):
        p_scr[:, BAND:BAND + N_CELLS, :] = P.reshape(BH, N_CELLS, DL)

    def shifted(o, sgn):
        d = BAND + sgn * o
        return p_scr[:, d:d + N_CELLS, :].reshape(RH, DL)

    onebdv = onebd[...]

    node = jnp.maximum(_dot(x_ref[...], emb_W[...]) + emb_b[...], 0.0)

    PQ = _dot(node, e0_W1ab[...])
    put(PQ[:, :DL])
    Q = PQ[:, DL:] + e0_b1[...]
    Wf, bf, o2 = Wfus[...], bfus[...], one2bd[...]
    Hs = jnp.zeros((RH, DL), jnp.float32)
    for gi, (o, sgn) in enumerate(GROUPS):
        h = jnp.maximum(shifted(o, sgn) + Q, 0.0)
        Hs = Hs + h
        EC = _dot(h, Wf) + bf
        e = EC[:, :DL]
        MM = _dot(jnp.concatenate([e, e * e], axis=1), o2)
        mw, msw = MM[:, :DL], MM[:, DL:]
        rw = jax.lax.rsqrt(msw - mw * mw + 1e-5)
        h_scr[gi] = EC[:, DL:] * rw
    agg = _dot(Hs, e0_W2[...]) + degb0[...]
    h = jnp.maximum(_dot(node, n0_W1a[...]) + _dot(agg, n0_W1b[...]) + n0_b1[...], 0.0)
    node = _ln(_dot(h, n0_W2[...]) + n0_b2[...], nn_g0[...], nn_b0[...], onebdv)

    PQ = _dot(node, e1_W1ab[...])
    put(PQ[:, :DL])
    Q = PQ[:, DL:] + e1_b1[...]
    Hs = jnp.zeros((RH, DL), jnp.float32)
    for gi, (o, sgn) in enumerate(GROUPS):
        Hs = Hs + jnp.maximum(shifted(o, sgn) + Q + h_scr[gi], 0.0)
    agg = _dot(Hs, e1_W2[...]) + degb1[...]
    h = jnp.maximum(_dot(node, n1_W1a[...]) + _dot(agg, n1_W1b[...]) + n1_b1[...], 0.0)
    node = _ln(_dot(h, n1_W2[...]) + n1_b2[...], nn_g1[...], nn_b1[...], onebdv)

    PQ = _dot(node, h_Wab[...])
    put(PQ[:, :DL])
    Qh = PQ[:, DL:] + h_b[...]
    oW = o_W[...]
    cols = []
    for o in range(1, BAND + 1):
        hid = jnp.maximum(shifted(o, 1) + Qh, 0.0)
        cols.append(_dot(hid, oW))
    out_ref[...] = jnp.tanh(jnp.concatenate(cols, axis=1) + o_b[...])


def _pair_select():
    sel = []
    for i in range(N_CELLS):
        for j in range(max(0, i - BAND), i):
            sel.append(j * BAND + (i - j - 1))
    return np.asarray(sel, dtype=np.int32)


_SEL = _pair_select()


def _bd(W):
    z = jnp.zeros_like(W)
    return jnp.concatenate(
        [jnp.concatenate([W, z], axis=1), jnp.concatenate([z, W], axis=1)],
        axis=0)


def _t2(v):
    return jnp.concatenate([v.reshape(1, -1), v.reshape(1, -1)], axis=1)


def kernel(observations, edge_index, params):
    B = observations.shape[0]
    CH = B // (2 * BH)
    x = observations.reshape(CH, 2, BH, N_CELLS, F_PER_CELL)
    x = x.transpose(0, 2, 3, 1, 4).reshape(CH * RH, 2 * F_PER_CELL)
    p = params
    g0, g1 = p['gnn'][0], p['gnn'][1]
    e0W1, e1W1 = g0['eW'][0], g1['eW'][0]
    n0W1, n1W1 = g0['nW'][0], g1['nW'][0]
    hW = p['hid_W']
    W1c = e1W1[2 * D:]
    W1cp = p['en_g'][0][:, None] * W1c
    W1cp_eff = W1cp - jnp.full((D, D), 1.0 / D, jnp.float32) * W1cp.sum(axis=0)[None, :]
    W2_0, b2_0 = g0['eW'][1], g0['eb'][1]
    onesD = jnp.full((D, D), 1.0 / D, jnp.float32)
    Wfus = jnp.concatenate([_bd(W2_0), _bd(W2_0 @ W1cp_eff)], axis=1)
    bfus = jnp.concatenate([_t2(b2_0), _t2(b2_0 @ W1cp_eff)], axis=1)
    e1_b1_eff = g1['eb'][0] + p['en_b'][0] @ W1c
    jc = np.arange(RH, dtype=np.int32) % N_CELLS
    deg = (np.minimum(jc, BAND) + np.minimum(N_CELLS - 1 - jc, BAND)
           ).astype(np.float32)[:, None]
    degc = jnp.asarray(deg)
    degb0 = degc * _t2(g0['eb'][1])
    degb1 = degc * _t2(g1['eb'][1])
    onebd = _bd(onesD)
    weights = [
        _bd(p['emb_W']), _t2(p['emb_b']),
        jnp.concatenate([_bd(e0W1[:D]), _bd(e0W1[D:])], axis=1),
        _t2(g0['eb'][0]), _bd(g0['eW'][1]), degb0,
        _bd(n0W1[:D]), _bd(n0W1[D:]), _t2(g0['nb'][0]), _bd(g0['nW'][1]),
        _t2(g0['nb'][1]),
        _t2(p['nn_g'][0]), _t2(p['nn_b'][0]),
        jnp.concatenate([_bd(e1W1[:D]), _bd(e1W1[D:2 * D])], axis=1),
        _t2(e1_b1_eff), _bd(g1['eW'][1]), degb1, Wfus, bfus,
        _bd(n1W1[:D]), _bd(n1W1[D:]), _t2(g1['nb'][0]), _bd(g1['nW'][1]),
        _t2(g1['nb'][1]),
        _t2(p['nn_g'][1]), _t2(p['nn_b'][1]),
        jnp.concatenate([_bd(hW[:D]), _bd(hW[D:])], axis=1),
        _t2(p['hid_b']), _bd(p['out_W']), p['out_b'].reshape(1, 1),
        onebd, _bd(onebd),
    ]
    out = pl.pallas_call(
        _fwd_kernel,
        grid=(CH,),
        in_specs=[pl.BlockSpec((RH, 2 * F_PER_CELL), lambda i: (i, 0))]
        + [pl.BlockSpec(w.shape, lambda i, nd=w.ndim: (0,) * nd) for w in weights],
        out_specs=pl.BlockSpec((RH, 2 * BAND), lambda i: (i, 0)),
        out_shape=jax.ShapeDtypeStruct((CH * RH, 2 * BAND), jnp.float32),
        scratch_shapes=[
            pltpu.VMEM((len(GROUPS), RH, 2 * D), jnp.float32),
            pltpu.VMEM((BH, N_CELLS + 2 * BAND, 2 * D), jnp.float32),
        ],
    )(x, *weights)
    out = out.reshape(CH, BH, N_CELLS, BAND, 2)
    out = out.transpose(0, 4, 1, 2, 3).reshape(B, N_CELLS * BAND)
    return out[:, _SEL]
